# baseline (speedup 1.0000x reference)
import functools

import jax
import jax.numpy as jnp
from jax import lax
from jax.experimental import pallas as pl
from jax.experimental.pallas import tpu as pltpu
from jax.experimental.pallas import tpu_sc as plsc

_N = 10000
_E = 160000
_DIN = 256
_DOUT = 256
_H = 8

_TILE_N = 1024
_GRID = 10
_NPAD = _TILE_N * _GRID

_NT = 16
_ET = _E // _NT
_K = 80
_NCH = _ET // _K
_NCH2 = _NCH // 2
_NVEC = _ET // 16
_NROW_T = _NPAD // _NT
_NEG = -3e38


def _tc_body(x_ref, wcat_ref, bcat_ref, atj_ref, abj_ref, tab_ref, stb_ref):
    xb = x_ref[...]
    tops = []
    bots = []
    for j in range(16):
        wh = jnp.dot(xb, wcat_ref[j], preferred_element_type=jnp.float32)
        wh = wh + bcat_ref[pl.ds(j, 1), :]
        tab_ref[j] = wh
        dn = (((1,), (1,)), ((), ()))
        tops.append(lax.dot_general(atj_ref[pl.ds(j, 1), :], wh, dn,
                                    preferred_element_type=jnp.float32))
        bots.append(lax.dot_general(abj_ref[pl.ds(j, 1), :], wh, dn,
                                    preferred_element_type=jnp.float32))
    topp = jnp.concatenate(tops, axis=0).reshape(8, 2, _TILE_N).sum(axis=1)
    botp = jnp.concatenate(bots, axis=0).reshape(8, 2, _TILE_N).sum(axis=1)
    stb_ref[...] = jnp.concatenate([topp, botp], axis=0)


def _tc_call(xp, wcat, bcat, atj, abj):
    return pl.pallas_call(
        _tc_body,
        grid=(_GRID,),
        in_specs=[
            pl.BlockSpec((_TILE_N, _DIN), lambda i: (i, 0)),
            pl.BlockSpec((16, _DIN, 128), lambda i: (0, 0, 0)),
            pl.BlockSpec((16, 128), lambda i: (0, 0)),
            pl.BlockSpec((16, 128), lambda i: (0, 0)),
            pl.BlockSpec((16, 128), lambda i: (0, 0)),
        ],
        out_specs=[
            pl.BlockSpec((16, _TILE_N, 128), lambda i: (0, i, 0)),
            pl.BlockSpec((16, _TILE_N), lambda i: (0, i)),
        ],
        out_shape=[
            jax.ShapeDtypeStruct((16, _NPAD, 128), jnp.float32),
            jax.ShapeDtypeStruct((16, _NPAD), jnp.float32),
        ],
    )(xp, wcat, bcat, atj, abj)


def _bfly_max(tmp, x):
    tmp[...] = x
    lane = lax.iota(jnp.int32, 16)
    for k in (8, 4, 2, 1):
        tmp[...] = jnp.maximum(tmp[...], plsc.load_gather(tmp, [lane ^ k]))
    return tmp[...]


def _bfly_sum(tmp, x):
    tmp[...] = x
    lane = lax.iota(jnp.int32, 16)
    for k in (8, 4, 2, 1):
        tmp[...] = tmp[...] + plsc.load_gather(tmp, [lane ^ k])
    return tmp[...]


def _sca_body(stb_hbm, rows_hbm, cols_hbm, gate_hbm, coef_hbm,
              row1, col1, stop_v, sbot_v, ev, redv, gate_v, vtmp,
              maxb, sumb):
    cid = lax.axis_index("c")
    sid = lax.axis_index("s")

    pltpu.sync_copy(rows_hbm.at[pl.ds(sid * _ET, _ET)], row1)
    pltpu.sync_copy(cols_hbm.at[pl.ds(sid * _ET, _ET)], col1)
    pltpu.sync_copy(gate_hbm, gate_v)

    lane = lax.iota(jnp.int32, 16)
    gmask = lane < _H
    gv = jnp.where(gmask, gate_v[...], jnp.float32(_NEG))
    gmx = _bfly_max(vtmp, gv)
    gexp = jnp.where(gmask, jnp.exp(gv - gmx), jnp.float32(0.0))
    gsum = _bfly_sum(vtmp, gexp)

    def head_body(hh, _):
        m = 2 * hh + cid
        pltpu.sync_copy(stb_hbm.at[pl.ds(m * _NPAD, _NPAD)], stop_v)
        pltpu.sync_copy(stb_hbm.at[pl.ds((m + _H) * _NPAD, _NPAD)], sbot_v)

        vtmp[...] = jnp.full((16,), _NEG, jnp.float32)

        def _p1(i, _1):
            ridx = row1[pl.ds(i * 16, 16)]
            cidx = col1[pl.ds(i * 16, 16)]
            s = (plsc.load_gather(stop_v, [ridx])
                 + plsc.load_gather(sbot_v, [cidx]))
            e = jnp.maximum(s, jnp.float32(0.01) * s)
            ev[pl.ds(i * 16, 16)] = e
            vtmp[...] = jnp.maximum(vtmp[...], e)
            return 0
        lax.fori_loop(0, _NVEC, _p1, 0)

        pltpu.sync_copy(vtmp, maxb.at[pl.ds(sid * 16, 16)])
        plsc.subcore_barrier()
        pltpu.sync_copy(maxb, redv)
        rmx = redv[pl.ds(0, 16)]
        for t in range(1, _NT):
            rmx = jnp.maximum(rmx, redv[pl.ds(t * 16, 16)])
        gmaxv = _bfly_max(vtmp, rmx)
        plsc.subcore_barrier()

        vtmp[...] = jnp.zeros((16,), jnp.float32)

        def _p2(i, _1):
            v = jnp.exp(ev[pl.ds(i * 16, 16)] - gmaxv)
            ev[pl.ds(i * 16, 16)] = v
            vtmp[...] = vtmp[...] + v
            return 0
        lax.fori_loop(0, _NVEC, _p2, 0)

        pltpu.sync_copy(vtmp, sumb.at[pl.ds(sid * 16, 16)])
        plsc.subcore_barrier()
        pltpu.sync_copy(sumb, redv)
        rsm = redv[pl.ds(0, 16)]
        for t in range(1, _NT):
            rsm = rsm + redv[pl.ds(t * 16, 16)]
        total = _bfly_sum(vtmp, rsm)
        plsc.subcore_barrier()

        gm = _bfly_sum(vtmp, jnp.where(lane == m, gexp, jnp.float32(0.0)))
        scalev = gm / gsum / total

        def _p3(i, _1):
            ev[pl.ds(i * 16, 16)] = ev[pl.ds(i * 16, 16)] * scalev
            return 0
        lax.fori_loop(0, _NVEC, _p3, 0)

        pltpu.sync_copy(ev, coef_hbm.at[pl.ds(m * _E + sid * _ET, _ET)])
        return 0

    lax.fori_loop(0, _H // 2, head_body, 0)


def _sca_call(stb, rows_w, cols_w, gate16):
    mesh = plsc.VectorSubcoreMesh(core_axis_name="c", subcore_axis_name="s")
    f = functools.partial(
        pl.kernel,
        out_type=jax.ShapeDtypeStruct((_H * _E,), jnp.float32),
        mesh=mesh,
        scratch_types=[
            pltpu.VMEM((_ET,), jnp.int32),
            pltpu.VMEM((_ET,), jnp.int32),
            pltpu.VMEM((_NPAD,), jnp.float32),
            pltpu.VMEM((_NPAD,), jnp.float32),
            pltpu.VMEM((_ET,), jnp.float32),
            pltpu.VMEM((256,), jnp.float32),
            pltpu.VMEM((16,), jnp.float32),
            pltpu.VMEM((16,), jnp.float32),
            pltpu.VMEM_SHARED((256,), jnp.float32),
            pltpu.VMEM_SHARED((256,), jnp.float32),
        ],
        compiler_params=pltpu.CompilerParams(needs_layout_passes=False),
    )(_sca_body)
    return f(stb, rows_w, cols_w, gate16)


def _scb_body(tab_hbm, coef_hbm, rows_hbm, cols_hbm, out_hbm,
              row3, col1, cb0, cb1, gb0, gb1, acc, gs0, gs1, cs0, cs1):
    cid = lax.axis_index("c")
    sid = lax.axis_index("s")

    pltpu.sync_copy(rows_hbm.at[sid], row3)
    pltpu.sync_copy(cols_hbm.at[pl.ds(sid * _ET, _ET)], col1)

    def _zb(i, _1):
        for w in range(8):
            gb0[i, pl.ds(w * 16, 16)] = jnp.zeros((16,), jnp.float32)
        return 0
    lax.fori_loop(0, _K, _zb, 0)
    for k in range(_NROW_T // _K):
        pltpu.sync_copy(gb0, acc.at[pl.ds(sid * _NROW_T + k * _K, _K)])
    plsc.subcore_barrier()

    def _scale(gb, cb):
        @plsc.parallel_loop(0, _K, unroll=8)
        def _rr(r):
            cv = cb[pl.ds(r, 16)]
            coeff = jnp.broadcast_to(cv[0], (16,))
            for w in range(8):
                gb[r, pl.ds(w * 16, 16)] = gb[r, pl.ds(w * 16, 16)] * coeff

    def head_body(m, _):
        jtab = 2 * m + cid
        tabm = tab_hbm.at[jtab]
        cbase = m * _E + sid * _ET

        def gsl(c):
            return col1.at[pl.ds(c * _K, _K)]

        def csl(c):
            return coef_hbm.at[pl.ds(cbase + c * _K, _K)]

        pltpu.async_copy(tabm.at[gsl(0)], gb0, gs0)
        pltpu.async_copy(csl(0), cb0.at[pl.ds(0, _K)], cs0)

        def _ch2(c2, _1):
            c0 = 2 * c2
            c1 = c0 + 1
            cn = jnp.where(c2 + 1 < _NCH2, c0 + 2, _NCH - 1)
            pltpu.async_copy(tabm.at[gsl(c1)], gb1, gs1)
            pltpu.async_copy(csl(c1), cb1.at[pl.ds(0, _K)], cs1)
            pltpu.make_async_copy(tabm.at[gsl(c0)], gb0, gs0).wait()
            pltpu.make_async_copy(csl(c0), cb0.at[pl.ds(0, _K)], cs0).wait()
            _scale(gb0, cb0)
            s0 = pltpu.async_copy(gb0, acc.at[row3.at[c0]], gs0, add=True)
            pltpu.make_async_copy(tabm.at[gsl(c1)], gb1, gs1).wait()
            pltpu.make_async_copy(csl(c1), cb1.at[pl.ds(0, _K)], cs1).wait()
            _scale(gb1, cb1)
            s1 = pltpu.async_copy(gb1, acc.at[row3.at[c1]], gs1, add=True)
            s0.wait()
            pltpu.async_copy(tabm.at[gsl(cn)], gb0, gs0)
            pltpu.async_copy(csl(cn), cb0.at[pl.ds(0, _K)], cs0)
            s1.wait()
            return 0
        lax.fori_loop(0, _NCH2, _ch2, 0)

        ct = _NCH - 1
        pltpu.make_async_copy(tabm.at[gsl(ct)], gb0, gs0).wait()
        pltpu.make_async_copy(csl(ct), cb0.at[pl.ds(0, _K)], cs0).wait()
        _scale(gb0, cb0)
        pltpu.async_copy(gb0, acc.at[row3.at[ct]], gs0, add=True).wait()
        return 0

    lax.fori_loop(0, _H, head_body, 0)
    plsc.subcore_barrier()

    pltpu.sync_copy(
        acc.at[pl.ds(sid * _NROW_T, _NROW_T)],
        out_hbm.at[pl.ds(sid * _NROW_T, _NROW_T), pl.ds(cid * 128, 128)])


def _scb_call(tab, coef, rows_r, cols_r):
    mesh = plsc.VectorSubcoreMesh(core_axis_name="c", subcore_axis_name="s")
    f = functools.partial(
        pl.kernel,
        out_type=jax.ShapeDtypeStruct((_NPAD, _DOUT), jnp.float32),
        mesh=mesh,
        scratch_types=[
            pltpu.VMEM((_NCH, _K), jnp.int32),
            pltpu.VMEM((_ET,), jnp.int32),
            pltpu.VMEM((_K + 16,), jnp.float32),
            pltpu.VMEM((_K + 16,), jnp.float32),
            pltpu.VMEM((_K, 128), jnp.float32),
            pltpu.VMEM((_K, 128), jnp.float32),
            pltpu.VMEM_SHARED((_NPAD, 128), jnp.float32),
            pltpu.SemaphoreType.DMA,
            pltpu.SemaphoreType.DMA,
            pltpu.SemaphoreType.DMA,
            pltpu.SemaphoreType.DMA,
        ],
        compiler_params=pltpu.CompilerParams(needs_layout_passes=False),
    )(_scb_body)
    return f(tab, coef, rows_r, cols_r)


def kernel(x, edge_index, W_w, W_b, a, gate):
    xp = jnp.pad(x, ((0, _NPAD - _N), (0, 0)))
    wcat = W_w.reshape(_H, _DIN, 2, 128).transpose(0, 2, 1, 3).reshape(16, _DIN, 128)
    bcat = W_b.reshape(16, 128)
    a2 = jnp.squeeze(a, axis=-1)
    atj = a2[:, :_DOUT].reshape(16, 128)
    abj = a2[:, _DOUT:].reshape(16, 128)
    ei = edge_index.astype(jnp.int32)
    rows_w = ei[0]
    cols_w = ei[1]
    rows_r = ei[0].reshape(_NT, _NCH, _K)
    gate16 = jnp.pad(gate.astype(jnp.float32), (0, 16 - _H))

    tab, stb = _tc_call(xp, wcat, bcat, atj, abj)
    stbf = stb.reshape(16 * _NPAD)
    coef = _sca_call(stbf, rows_w, cols_w, gate16)
    res = _scb_call(tab, coef, rows_r, cols_w)
    return res[:_N]


# speedup vs baseline: 1.0003x; 1.0003x over previous
import functools

import jax
import jax.numpy as jnp
from jax import lax
from jax.experimental import pallas as pl
from jax.experimental.pallas import tpu as pltpu
from jax.experimental.pallas import tpu_sc as plsc

_N = 10000
_E = 160000
_DIN = 256
_DOUT = 256
_H = 8

_TILE_N = 1024
_GRID = 10
_NPAD = _TILE_N * _GRID

_NT = 16
_ET = _E // _NT
_K = 80
_NCH = _ET // _K
_NCH2 = _NCH // 2
_NVEC = _ET // 16
_NROW_T = _NPAD // _NT
_NEG = -3e38


def _tc_body(x_ref, wcat_ref, bcat_ref, atj_ref, abj_ref, tab_ref, stb_ref):
    xb = x_ref[...]
    tops = []
    bots = []
    for j in range(16):
        wh = jnp.dot(xb, wcat_ref[j], preferred_element_type=jnp.float32)
        wh = wh + bcat_ref[pl.ds(j, 1), :]
        tab_ref[j] = wh
        dn = (((1,), (1,)), ((), ()))
        tops.append(lax.dot_general(atj_ref[pl.ds(j, 1), :], wh, dn,
                                    preferred_element_type=jnp.float32))
        bots.append(lax.dot_general(abj_ref[pl.ds(j, 1), :], wh, dn,
                                    preferred_element_type=jnp.float32))
    topp = jnp.concatenate(tops, axis=0).reshape(8, 2, _TILE_N).sum(axis=1)
    botp = jnp.concatenate(bots, axis=0).reshape(8, 2, _TILE_N).sum(axis=1)
    stb_ref[...] = jnp.concatenate([topp, botp], axis=0)


def _tc_call(xp, wcat, bcat, atj, abj):
    return pl.pallas_call(
        _tc_body,
        grid=(_GRID,),
        in_specs=[
            pl.BlockSpec((_TILE_N, _DIN), lambda i: (i, 0)),
            pl.BlockSpec((16, _DIN, 128), lambda i: (0, 0, 0)),
            pl.BlockSpec((16, 128), lambda i: (0, 0)),
            pl.BlockSpec((16, 128), lambda i: (0, 0)),
            pl.BlockSpec((16, 128), lambda i: (0, 0)),
        ],
        out_specs=[
            pl.BlockSpec((16, _TILE_N, 128), lambda i: (0, i, 0)),
            pl.BlockSpec((16, _TILE_N), lambda i: (0, i)),
        ],
        out_shape=[
            jax.ShapeDtypeStruct((16, _NPAD, 128), jnp.float32),
            jax.ShapeDtypeStruct((16, _NPAD), jnp.float32),
        ],
    )(xp, wcat, bcat, atj, abj)


def _bfly_max(tmp, x):
    tmp[...] = x
    lane = lax.iota(jnp.int32, 16)
    for k in (8, 4, 2, 1):
        tmp[...] = jnp.maximum(tmp[...], plsc.load_gather(tmp, [lane ^ k]))
    return tmp[...]


def _bfly_sum(tmp, x):
    tmp[...] = x
    lane = lax.iota(jnp.int32, 16)
    for k in (8, 4, 2, 1):
        tmp[...] = tmp[...] + plsc.load_gather(tmp, [lane ^ k])
    return tmp[...]


def _sca_body(stb_hbm, rows_hbm, cols_hbm, gate_hbm, coef_hbm,
              row1, col1, stop_v, sbot_v, ev, redv, gate_v, vtmp,
              maxb, sumb):
    cid = lax.axis_index("c")
    sid = lax.axis_index("s")

    pltpu.sync_copy(rows_hbm.at[pl.ds(sid * _ET, _ET)], row1)
    pltpu.sync_copy(cols_hbm.at[pl.ds(sid * _ET, _ET)], col1)
    pltpu.sync_copy(gate_hbm, gate_v)

    lane = lax.iota(jnp.int32, 16)
    gmask = lane < _H
    gv = jnp.where(gmask, gate_v[...], jnp.float32(_NEG))
    gmx = _bfly_max(vtmp, gv)
    gexp = jnp.where(gmask, jnp.exp(gv - gmx), jnp.float32(0.0))
    gsum = _bfly_sum(vtmp, gexp)

    def head_body(hh, _):
        m = 2 * hh + cid
        pltpu.sync_copy(stb_hbm.at[pl.ds(m * _NPAD, _NPAD)], stop_v)
        pltpu.sync_copy(stb_hbm.at[pl.ds((m + _H) * _NPAD, _NPAD)], sbot_v)

        vtmp[...] = jnp.full((16,), _NEG, jnp.float32)

        def _p1(i, _1):
            ridx = row1[pl.ds(i * 16, 16)]
            cidx = col1[pl.ds(i * 16, 16)]
            s = (plsc.load_gather(stop_v, [ridx])
                 + plsc.load_gather(sbot_v, [cidx]))
            e = jnp.maximum(s, jnp.float32(0.01) * s)
            ev[pl.ds(i * 16, 16)] = e
            vtmp[...] = jnp.maximum(vtmp[...], e)
            return 0
        lax.fori_loop(0, _NVEC, _p1, 0)

        pltpu.sync_copy(vtmp, maxb.at[pl.ds(sid * 16, 16)])
        plsc.subcore_barrier()
        pltpu.sync_copy(maxb, redv)
        rmx = redv[pl.ds(0, 16)]
        for t in range(1, _NT):
            rmx = jnp.maximum(rmx, redv[pl.ds(t * 16, 16)])
        gmaxv = _bfly_max(vtmp, rmx)
        plsc.subcore_barrier()

        vtmp[...] = jnp.zeros((16,), jnp.float32)

        def _p2(i, _1):
            v = jnp.exp(ev[pl.ds(i * 16, 16)] - gmaxv)
            ev[pl.ds(i * 16, 16)] = v
            vtmp[...] = vtmp[...] + v
            return 0
        lax.fori_loop(0, _NVEC, _p2, 0)

        pltpu.sync_copy(vtmp, sumb.at[pl.ds(sid * 16, 16)])
        plsc.subcore_barrier()
        pltpu.sync_copy(sumb, redv)
        rsm = redv[pl.ds(0, 16)]
        for t in range(1, _NT):
            rsm = rsm + redv[pl.ds(t * 16, 16)]
        total = _bfly_sum(vtmp, rsm)
        plsc.subcore_barrier()

        gm = _bfly_sum(vtmp, jnp.where(lane == m, gexp, jnp.float32(0.0)))
        scalev = gm / gsum / total

        def _p3(i, _1):
            ev[pl.ds(i * 16, 16)] = ev[pl.ds(i * 16, 16)] * scalev
            return 0
        lax.fori_loop(0, _NVEC, _p3, 0)

        pltpu.sync_copy(ev, coef_hbm.at[pl.ds(m * _E + sid * _ET, _ET)])
        return 0

    lax.fori_loop(0, _H // 2, head_body, 0)


def _sca_call(stb, rows_w, cols_w, gate16):
    mesh = plsc.VectorSubcoreMesh(core_axis_name="c", subcore_axis_name="s")
    f = functools.partial(
        pl.kernel,
        out_type=jax.ShapeDtypeStruct((_H * _E,), jnp.float32),
        mesh=mesh,
        scratch_types=[
            pltpu.VMEM((_ET,), jnp.int32),
            pltpu.VMEM((_ET,), jnp.int32),
            pltpu.VMEM((_NPAD,), jnp.float32),
            pltpu.VMEM((_NPAD,), jnp.float32),
            pltpu.VMEM((_ET,), jnp.float32),
            pltpu.VMEM((256,), jnp.float32),
            pltpu.VMEM((16,), jnp.float32),
            pltpu.VMEM((16,), jnp.float32),
            pltpu.VMEM_SHARED((256,), jnp.float32),
            pltpu.VMEM_SHARED((256,), jnp.float32),
        ],
        compiler_params=pltpu.CompilerParams(needs_layout_passes=False),
    )(_sca_body)
    return f(stb, rows_w, cols_w, gate16)


def _scb_body(tab_hbm, coef_hbm, rows_hbm, cols_hbm, out_hbm,
              row3, col1, cb0, cb1, gb0, gb1, acc, gs0, gs1, cs0, cs1):
    cid = lax.axis_index("c")
    sid = lax.axis_index("s")

    pltpu.sync_copy(rows_hbm.at[sid], row3)
    pltpu.sync_copy(cols_hbm.at[pl.ds(sid * _ET, _ET)], col1)

    def _zb(i, _1):
        for w in range(8):
            gb0[i, pl.ds(w * 16, 16)] = jnp.zeros((16,), jnp.float32)
        return 0
    lax.fori_loop(0, _K, _zb, 0)
    for k in range(_NROW_T // _K):
        pltpu.sync_copy(gb0, acc.at[pl.ds(sid * _NROW_T + k * _K, _K)])
    plsc.subcore_barrier()

    def _scale(gb, cb):
        @plsc.parallel_loop(0, _K, unroll=4)
        def _rr(r):
            cv = cb[pl.ds(r, 16)]
            coeff = jnp.broadcast_to(cv[0], (16,))
            for w in range(8):
                gb[r, pl.ds(w * 16, 16)] = gb[r, pl.ds(w * 16, 16)] * coeff

    def head_body(m, _):
        jtab = 2 * m + cid
        tabm = tab_hbm.at[jtab]
        cbase = m * _E + sid * _ET

        def gsl(c):
            return col1.at[pl.ds(c * _K, _K)]

        def csl(c):
            return coef_hbm.at[pl.ds(cbase + c * _K, _K)]

        pltpu.async_copy(tabm.at[gsl(0)], gb0, gs0)
        pltpu.async_copy(csl(0), cb0.at[pl.ds(0, _K)], cs0)

        def _ch2(c2, _1):
            c0 = 2 * c2
            c1 = c0 + 1
            cn = jnp.where(c2 + 1 < _NCH2, c0 + 2, _NCH - 1)
            pltpu.async_copy(tabm.at[gsl(c1)], gb1, gs1)
            pltpu.async_copy(csl(c1), cb1.at[pl.ds(0, _K)], cs1)
            pltpu.make_async_copy(tabm.at[gsl(c0)], gb0, gs0).wait()
            pltpu.make_async_copy(csl(c0), cb0.at[pl.ds(0, _K)], cs0).wait()
            _scale(gb0, cb0)
            s0 = pltpu.async_copy(gb0, acc.at[row3.at[c0]], gs0, add=True)
            pltpu.make_async_copy(tabm.at[gsl(c1)], gb1, gs1).wait()
            pltpu.make_async_copy(csl(c1), cb1.at[pl.ds(0, _K)], cs1).wait()
            _scale(gb1, cb1)
            s1 = pltpu.async_copy(gb1, acc.at[row3.at[c1]], gs1, add=True)
            s0.wait()
            pltpu.async_copy(tabm.at[gsl(cn)], gb0, gs0)
            pltpu.async_copy(csl(cn), cb0.at[pl.ds(0, _K)], cs0)
            s1.wait()
            return 0
        lax.fori_loop(0, _NCH2, _ch2, 0)

        ct = _NCH - 1
        pltpu.make_async_copy(tabm.at[gsl(ct)], gb0, gs0).wait()
        pltpu.make_async_copy(csl(ct), cb0.at[pl.ds(0, _K)], cs0).wait()
        _scale(gb0, cb0)
        pltpu.async_copy(gb0, acc.at[row3.at[ct]], gs0, add=True).wait()
        return 0

    lax.fori_loop(0, _H, head_body, 0)
    plsc.subcore_barrier()

    pltpu.sync_copy(
        acc.at[pl.ds(sid * _NROW_T, _NROW_T)],
        out_hbm.at[pl.ds(sid * _NROW_T, _NROW_T), pl.ds(cid * 128, 128)])


def _scb_call(tab, coef, rows_r, cols_r):
    mesh = plsc.VectorSubcoreMesh(core_axis_name="c", subcore_axis_name="s")
    f = functools.partial(
        pl.kernel,
        out_type=jax.ShapeDtypeStruct((_NPAD, _DOUT), jnp.float32),
        mesh=mesh,
        scratch_types=[
            pltpu.VMEM((_NCH, _K), jnp.int32),
            pltpu.VMEM((_ET,), jnp.int32),
            pltpu.VMEM((_K + 16,), jnp.float32),
            pltpu.VMEM((_K + 16,), jnp.float32),
            pltpu.VMEM((_K, 128), jnp.float32),
            pltpu.VMEM((_K, 128), jnp.float32),
            pltpu.VMEM_SHARED((_NPAD, 128), jnp.float32),
            pltpu.SemaphoreType.DMA,
            pltpu.SemaphoreType.DMA,
            pltpu.SemaphoreType.DMA,
            pltpu.SemaphoreType.DMA,
        ],
        compiler_params=pltpu.CompilerParams(needs_layout_passes=False),
    )(_scb_body)
    return f(tab, coef, rows_r, cols_r)


def kernel(x, edge_index, W_w, W_b, a, gate):
    xp = jnp.pad(x, ((0, _NPAD - _N), (0, 0)))
    wcat = W_w.reshape(_H, _DIN, 2, 128).transpose(0, 2, 1, 3).reshape(16, _DIN, 128)
    bcat = W_b.reshape(16, 128)
    a2 = jnp.squeeze(a, axis=-1)
    atj = a2[:, :_DOUT].reshape(16, 128)
    abj = a2[:, _DOUT:].reshape(16, 128)
    ei = edge_index.astype(jnp.int32)
    rows_w = ei[0]
    cols_w = ei[1]
    rows_r = ei[0].reshape(_NT, _NCH, _K)
    gate16 = jnp.pad(gate.astype(jnp.float32), (0, 16 - _H))

    tab, stb = _tc_call(xp, wcat, bcat, atj, abj)
    stbf = stb.reshape(16 * _NPAD)
    coef = _sca_call(stbf, rows_w, cols_w, gate16)
    res = _scb_call(tab, coef, rows_r, cols_w)
    return res[:_N]


# speedup vs baseline: 1.0663x; 1.0659x over previous
import functools

import jax
import jax.numpy as jnp
from jax import lax
from jax.experimental import pallas as pl
from jax.experimental.pallas import tpu as pltpu
from jax.experimental.pallas import tpu_sc as plsc

_N = 10000
_E = 160000
_DIN = 256
_DOUT = 256
_H = 8

_TILE_N = 1024
_GRID = 10
_NPAD = _TILE_N * _GRID

_NT = 16
_ET = _E // _NT
_K = 80
_NCH = _ET // _K
_NCH2 = _NCH // 2
_NVEC = _ET // 16
_NROW_T = _NPAD // _NT
_NEG = -3e38


def _tc_body(x_ref, wcat_ref, bcat_ref, atj_ref, abj_ref, tab_ref, stb_ref):
    xb = x_ref[...]
    tops = []
    bots = []
    for j in range(16):
        wh = jnp.dot(xb, wcat_ref[j], preferred_element_type=jnp.float32)
        wh = wh + bcat_ref[pl.ds(j, 1), :]
        tab_ref[j] = wh
        dn = (((1,), (1,)), ((), ()))
        tops.append(lax.dot_general(atj_ref[pl.ds(j, 1), :], wh, dn,
                                    preferred_element_type=jnp.float32))
        bots.append(lax.dot_general(abj_ref[pl.ds(j, 1), :], wh, dn,
                                    preferred_element_type=jnp.float32))
    topp = jnp.concatenate(tops, axis=0).reshape(8, 2, _TILE_N).sum(axis=1)
    botp = jnp.concatenate(bots, axis=0).reshape(8, 2, _TILE_N).sum(axis=1)
    stb_ref[...] = jnp.concatenate([topp, botp], axis=0)


def _tc_call(xp, wcat, bcat, atj, abj):
    return pl.pallas_call(
        _tc_body,
        grid=(_GRID,),
        in_specs=[
            pl.BlockSpec((_TILE_N, _DIN), lambda i: (i, 0)),
            pl.BlockSpec((16, _DIN, 128), lambda i: (0, 0, 0)),
            pl.BlockSpec((16, 128), lambda i: (0, 0)),
            pl.BlockSpec((16, 128), lambda i: (0, 0)),
            pl.BlockSpec((16, 128), lambda i: (0, 0)),
        ],
        out_specs=[
            pl.BlockSpec((16, _TILE_N, 128), lambda i: (0, i, 0)),
            pl.BlockSpec((16, _TILE_N), lambda i: (0, i)),
        ],
        out_shape=[
            jax.ShapeDtypeStruct((16, _NPAD, 128), jnp.float32),
            jax.ShapeDtypeStruct((16, _NPAD), jnp.float32),
        ],
    )(xp, wcat, bcat, atj, abj)


def _bfly_max(tmp, x):
    tmp[...] = x
    lane = lax.iota(jnp.int32, 16)
    for k in (8, 4, 2, 1):
        tmp[...] = jnp.maximum(tmp[...], plsc.load_gather(tmp, [lane ^ k]))
    return tmp[...]


def _bfly_sum(tmp, x):
    tmp[...] = x
    lane = lax.iota(jnp.int32, 16)
    for k in (8, 4, 2, 1):
        tmp[...] = tmp[...] + plsc.load_gather(tmp, [lane ^ k])
    return tmp[...]


def _sca_body(stb_hbm, rows_hbm, cols_hbm, gate_hbm, coef_hbm,
              row1, col1, stop_v, sbot_v, ev, redv, gate_v, vtmp,
              maxb, sumb):
    cid = lax.axis_index("c")
    sid = lax.axis_index("s")

    pltpu.sync_copy(rows_hbm.at[pl.ds(sid * _ET, _ET)], row1)
    pltpu.sync_copy(cols_hbm.at[pl.ds(sid * _ET, _ET)], col1)
    pltpu.sync_copy(gate_hbm, gate_v)

    lane = lax.iota(jnp.int32, 16)
    gmask = lane < _H
    gv = jnp.where(gmask, gate_v[...], jnp.float32(_NEG))
    gmx = _bfly_max(vtmp, gv)
    gexp = jnp.where(gmask, jnp.exp(gv - gmx), jnp.float32(0.0))
    gsum = _bfly_sum(vtmp, gexp)

    def head_body(hh, _):
        m = 2 * hh + cid
        pltpu.sync_copy(stb_hbm.at[pl.ds(m * _NPAD, _NPAD)], stop_v)
        pltpu.sync_copy(stb_hbm.at[pl.ds((m + _H) * _NPAD, _NPAD)], sbot_v)

        @plsc.parallel_loop(0, _NVEC, unroll=4,
                            carry=jnp.full((16,), _NEG, jnp.float32))
        def _p1(i, mx):
            ridx = row1[pl.ds(i * 16, 16)]
            cidx = col1[pl.ds(i * 16, 16)]
            s = (plsc.load_gather(stop_v, [ridx])
                 + plsc.load_gather(sbot_v, [cidx]))
            e = jnp.maximum(s, jnp.float32(0.01) * s)
            ev[pl.ds(i * 16, 16)] = e
            return jnp.maximum(mx, e)
        vtmp[...] = _p1

        pltpu.sync_copy(vtmp, maxb.at[pl.ds(sid * 16, 16)])
        plsc.subcore_barrier()
        pltpu.sync_copy(maxb, redv)
        rmx = redv[pl.ds(0, 16)]
        for t in range(1, _NT):
            rmx = jnp.maximum(rmx, redv[pl.ds(t * 16, 16)])
        gmaxv = _bfly_max(vtmp, rmx)
        plsc.subcore_barrier()

        @plsc.parallel_loop(0, _NVEC, unroll=4,
                            carry=jnp.zeros((16,), jnp.float32))
        def _p2(i, sm):
            v = jnp.exp(ev[pl.ds(i * 16, 16)] - gmaxv)
            ev[pl.ds(i * 16, 16)] = v
            return sm + v
        vtmp[...] = _p2

        pltpu.sync_copy(vtmp, sumb.at[pl.ds(sid * 16, 16)])
        plsc.subcore_barrier()
        pltpu.sync_copy(sumb, redv)
        rsm = redv[pl.ds(0, 16)]
        for t in range(1, _NT):
            rsm = rsm + redv[pl.ds(t * 16, 16)]
        total = _bfly_sum(vtmp, rsm)
        plsc.subcore_barrier()

        gm = _bfly_sum(vtmp, jnp.where(lane == m, gexp, jnp.float32(0.0)))
        scalev = gm / gsum / total

        @plsc.parallel_loop(0, _NVEC, unroll=4)
        def _p3(i):
            ev[pl.ds(i * 16, 16)] = ev[pl.ds(i * 16, 16)] * scalev

        pltpu.sync_copy(ev, coef_hbm.at[pl.ds(m * _E + sid * _ET, _ET)])
        return 0

    lax.fori_loop(0, _H // 2, head_body, 0)


def _sca_call(stb, rows_w, cols_w, gate16):
    mesh = plsc.VectorSubcoreMesh(core_axis_name="c", subcore_axis_name="s")
    f = functools.partial(
        pl.kernel,
        out_type=jax.ShapeDtypeStruct((_H * _E,), jnp.float32),
        mesh=mesh,
        scratch_types=[
            pltpu.VMEM((_ET,), jnp.int32),
            pltpu.VMEM((_ET,), jnp.int32),
            pltpu.VMEM((_NPAD,), jnp.float32),
            pltpu.VMEM((_NPAD,), jnp.float32),
            pltpu.VMEM((_ET,), jnp.float32),
            pltpu.VMEM((256,), jnp.float32),
            pltpu.VMEM((16,), jnp.float32),
            pltpu.VMEM((16,), jnp.float32),
            pltpu.VMEM_SHARED((256,), jnp.float32),
            pltpu.VMEM_SHARED((256,), jnp.float32),
        ],
        compiler_params=pltpu.CompilerParams(needs_layout_passes=False),
    )(_sca_body)
    return f(stb, rows_w, cols_w, gate16)


def _scb_body(tab_hbm, coef_hbm, rows_hbm, cols_hbm, out_hbm,
              row3, col1, cb0, cb1, gb0, gb1, acc, gs0, gs1, cs0, cs1):
    cid = lax.axis_index("c")
    sid = lax.axis_index("s")

    pltpu.sync_copy(rows_hbm.at[sid], row3)
    pltpu.sync_copy(cols_hbm.at[pl.ds(sid * _ET, _ET)], col1)

    def _zb(i, _1):
        for w in range(8):
            gb0[i, pl.ds(w * 16, 16)] = jnp.zeros((16,), jnp.float32)
        return 0
    lax.fori_loop(0, _K, _zb, 0)
    for k in range(_NROW_T // _K):
        pltpu.sync_copy(gb0, acc.at[pl.ds(sid * _NROW_T + k * _K, _K)])
    plsc.subcore_barrier()

    def _scale(gb, cb):
        @plsc.parallel_loop(0, _K, unroll=4)
        def _rr(r):
            cv = cb[pl.ds(r, 16)]
            coeff = jnp.broadcast_to(cv[0], (16,))
            for w in range(8):
                gb[r, pl.ds(w * 16, 16)] = gb[r, pl.ds(w * 16, 16)] * coeff

    def head_body(m, _):
        jtab = 2 * m + cid
        tabm = tab_hbm.at[jtab]
        cbase = m * _E + sid * _ET

        def gsl(c):
            return col1.at[pl.ds(c * _K, _K)]

        def csl(c):
            return coef_hbm.at[pl.ds(cbase + c * _K, _K)]

        pltpu.async_copy(tabm.at[gsl(0)], gb0, gs0)
        pltpu.async_copy(csl(0), cb0.at[pl.ds(0, _K)], cs0)

        def _ch2(c2, _1):
            c0 = 2 * c2
            c1 = c0 + 1
            cn = jnp.where(c2 + 1 < _NCH2, c0 + 2, _NCH - 1)
            pltpu.async_copy(tabm.at[gsl(c1)], gb1, gs1)
            pltpu.async_copy(csl(c1), cb1.at[pl.ds(0, _K)], cs1)
            pltpu.make_async_copy(tabm.at[gsl(c0)], gb0, gs0).wait()
            pltpu.make_async_copy(csl(c0), cb0.at[pl.ds(0, _K)], cs0).wait()
            _scale(gb0, cb0)
            s0 = pltpu.async_copy(gb0, acc.at[row3.at[c0]], gs0, add=True)
            pltpu.make_async_copy(tabm.at[gsl(c1)], gb1, gs1).wait()
            pltpu.make_async_copy(csl(c1), cb1.at[pl.ds(0, _K)], cs1).wait()
            _scale(gb1, cb1)
            s1 = pltpu.async_copy(gb1, acc.at[row3.at[c1]], gs1, add=True)
            s0.wait()
            pltpu.async_copy(tabm.at[gsl(cn)], gb0, gs0)
            pltpu.async_copy(csl(cn), cb0.at[pl.ds(0, _K)], cs0)
            s1.wait()
            return 0
        lax.fori_loop(0, _NCH2, _ch2, 0)

        ct = _NCH - 1
        pltpu.make_async_copy(tabm.at[gsl(ct)], gb0, gs0).wait()
        pltpu.make_async_copy(csl(ct), cb0.at[pl.ds(0, _K)], cs0).wait()
        _scale(gb0, cb0)
        pltpu.async_copy(gb0, acc.at[row3.at[ct]], gs0, add=True).wait()
        return 0

    lax.fori_loop(0, _H, head_body, 0)
    plsc.subcore_barrier()

    pltpu.sync_copy(
        acc.at[pl.ds(sid * _NROW_T, _NROW_T)],
        out_hbm.at[pl.ds(sid * _NROW_T, _NROW_T), pl.ds(cid * 128, 128)])


def _scb_call(tab, coef, rows_r, cols_r):
    mesh = plsc.VectorSubcoreMesh(core_axis_name="c", subcore_axis_name="s")
    f = functools.partial(
        pl.kernel,
        out_type=jax.ShapeDtypeStruct((_NPAD, _DOUT), jnp.float32),
        mesh=mesh,
        scratch_types=[
            pltpu.VMEM((_NCH, _K), jnp.int32),
            pltpu.VMEM((_ET,), jnp.int32),
            pltpu.VMEM((_K + 16,), jnp.float32),
            pltpu.VMEM((_K + 16,), jnp.float32),
            pltpu.VMEM((_K, 128), jnp.float32),
            pltpu.VMEM((_K, 128), jnp.float32),
            pltpu.VMEM_SHARED((_NPAD, 128), jnp.float32),
            pltpu.SemaphoreType.DMA,
            pltpu.SemaphoreType.DMA,
            pltpu.SemaphoreType.DMA,
            pltpu.SemaphoreType.DMA,
        ],
        compiler_params=pltpu.CompilerParams(needs_layout_passes=False),
    )(_scb_body)
    return f(tab, coef, rows_r, cols_r)


def kernel(x, edge_index, W_w, W_b, a, gate):
    xp = jnp.pad(x, ((0, _NPAD - _N), (0, 0)))
    wcat = W_w.reshape(_H, _DIN, 2, 128).transpose(0, 2, 1, 3).reshape(16, _DIN, 128)
    bcat = W_b.reshape(16, 128)
    a2 = jnp.squeeze(a, axis=-1)
    atj = a2[:, :_DOUT].reshape(16, 128)
    abj = a2[:, _DOUT:].reshape(16, 128)
    ei = edge_index.astype(jnp.int32)
    rows_w = ei[0]
    cols_w = ei[1]
    rows_r = ei[0].reshape(_NT, _NCH, _K)
    gate16 = jnp.pad(gate.astype(jnp.float32), (0, 16 - _H))

    tab, stb = _tc_call(xp, wcat, bcat, atj, abj)
    stbf = stb.reshape(16 * _NPAD)
    coef = _sca_call(stbf, rows_w, cols_w, gate16)
    res = _scb_call(tab, coef, rows_r, cols_w)
    return res[:_N]
